# baseline (speedup 1.0000x reference)
import functools

import jax
import jax.numpy as jnp
from jax import lax
from jax.experimental import pallas as pl
from jax.experimental.pallas import tpu as pltpu
from jax.experimental.pallas import tpu_sc as plsc

_BATCH = 16384
_D = 32
_H1 = 64
_H2 = 32
_NCLS = 2
_NC = 2
_NS = 16
_NW = _NC * _NS
_G = 8
_LANES = 128
_BM = 2048

_sc_mesh = plsc.VectorSubcoreMesh(core_axis_name="c", subcore_axis_name="s")


def _make_gather(batch):
    bpw = batch // _NW
    ngrp = bpw // _G
    ntri = (ngrp - 1) // 3

    @functools.partial(
        pl.kernel,
        mesh=_sc_mesh,
        out_type=(
            jax.ShapeDtypeStruct((_D, batch), jnp.float32),
            jax.ShapeDtypeStruct((_D, batch), jnp.float32),
        ),
        scratch_types=[
            pltpu.VMEM((bpw + 16,), jnp.int32),
            pltpu.VMEM((3 * _G, _D, _LANES), jnp.float32),
            pltpu.VMEM((_D, bpw), jnp.float32),
            pltpu.SemaphoreType.DMA,
            pltpu.SemaphoreType.DMA,
        ],
        compiler_params=pltpu.CompilerParams(use_tc_tiling_on_sc=True,
                                             needs_layout_passes=False),
    )
    def _sc_gather(ut_t, it_t, uidx, iidx, ue_out, ie_out,
                   idx_v, blocks, out_v, fsem, ssem):
        wid = lax.axis_index("s") * _NC + lax.axis_index("c")
        base = wid * bpw
        rows_lo = jnp.arange(16, dtype=jnp.int32)
        rows_hi = rows_lo + 16

        def _fire_one(tab, r, slot):
            start = pl.multiple_of((r >> 7) * _LANES, _LANES)
            pltpu.make_async_copy(
                tab.at[:, pl.ds(start, _LANES)], blocks.at[slot], fsem
            ).start()

        def _extract_one(tab, r, j, slot):
            pltpu.make_async_copy(
                tab.at[:, pl.ds(0, _LANES)], blocks.at[slot], fsem
            ).wait()
            lvec = jnp.full((16,), r & 127, dtype=jnp.int32)
            jvec = jnp.full((16,), j, dtype=jnp.int32)
            blk = blocks.at[slot]
            v0 = plsc.load_gather(blk, [rows_lo, lvec])
            v1 = plsc.load_gather(blk, [rows_hi, lvec])
            plsc.store_scatter(out_v, [rows_lo, jvec], v0)
            plsc.store_scatter(out_v, [rows_hi, jvec], v1)

        def _run_table(tab, idx_hbm, out_hbm):
            dst = idx_v.at[pl.ds(0, bpw)]
            pltpu.make_async_copy(idx_hbm.at[pl.ds(base, bpw)], dst, ssem).start()
            pltpu.make_async_copy(idx_hbm.at[pl.ds(base, bpw)], dst, ssem).wait()

            def _fire_group(g, slot0):
                iv = idx_v[pl.ds(g * _G, 16)]
                for kk in range(_G):
                    _fire_one(tab, iv[kk], slot0 + kk)

            def _extract_group(g, slot0):
                iv = idx_v[pl.ds(g * _G, 16)]
                for kk in range(_G):
                    _extract_one(tab, iv[kk], g * _G + kk, slot0 + kk)

            _fire_group(0, 0)
            _fire_group(1, _G)

            def _body(t, carry):
                g = 3 * t
                _extract_group(g, 0)

                @pl.when(g + 2 <= ngrp - 1)
                def _():
                    _fire_group(g + 2, 2 * _G)

                _extract_group(g + 1, _G)

                @pl.when(g + 3 <= ngrp - 1)
                def _():
                    _fire_group(g + 3, 0)

                _extract_group(g + 2, 2 * _G)

                @pl.when(g + 4 <= ngrp - 1)
                def _():
                    _fire_group(g + 4, _G)

                return carry

            lax.fori_loop(0, ntri, _body, 0)
            for gg in range(3 * ntri, ngrp):
                _extract_group(gg, (gg % 3) * _G)
            pltpu.sync_copy(out_v, out_hbm.at[:, pl.ds(base, bpw)])

        _run_table(ut_t, uidx, ue_out)
        _run_table(it_t, iidx, ie_out)

    return _sc_gather


def _mlp_body(ue, ie, w1u, w1i, b1, w2, b2, w3, b3, out):
    x = jnp.dot(w1u[...], ue[...], preferred_element_type=jnp.float32)
    x = x + jnp.dot(w1i[...], ie[...], preferred_element_type=jnp.float32)
    x = jnp.maximum(x + b1[...], 0.0)
    x = jnp.maximum(jnp.dot(w2[...], x, preferred_element_type=jnp.float32) + b2[...], 0.0)
    x = jnp.maximum(jnp.dot(w3[...], x, preferred_element_type=jnp.float32) + b3[...], 0.0)
    out[...] = x


def _make_mlp(batch):
    return pl.pallas_call(
        _mlp_body,
        grid=(batch // _BM,),
        in_specs=[
            pl.BlockSpec((_D, _BM), lambda i: (0, i)),
            pl.BlockSpec((_D, _BM), lambda i: (0, i)),
            pl.BlockSpec((_H1, _D), lambda i: (0, 0)),
            pl.BlockSpec((_H1, _D), lambda i: (0, 0)),
            pl.BlockSpec((_H1, 1), lambda i: (0, 0)),
            pl.BlockSpec((_H2, _H1), lambda i: (0, 0)),
            pl.BlockSpec((_H2, 1), lambda i: (0, 0)),
            pl.BlockSpec((_NCLS, _H2), lambda i: (0, 0)),
            pl.BlockSpec((_NCLS, 1), lambda i: (0, 0)),
        ],
        out_specs=pl.BlockSpec((_NCLS, _BM), lambda i: (0, i)),
        out_shape=jax.ShapeDtypeStruct((_NCLS, batch), jnp.float32),
    )


_HALF = _BATCH // 2
_gather_half = _make_gather(_HALF)
_mlp_half = _make_mlp(_HALF)


def kernel(user_input, item_input, user_table, item_table, W1, b1, W2, b2, W3, b3):
    ut_t = user_table.T
    it_t = item_table.T
    ui = user_input.astype(jnp.int32)
    ii = item_input.astype(jnp.int32)
    w1t = W1.T
    w1u, w1i = w1t[:, :_D], w1t[:, _D:]
    b1c = b1.reshape(_H1, 1)
    w2t = W2.T
    b2c = b2.reshape(_H2, 1)
    w3t = W3.T
    b3c = b3.reshape(_NCLS, 1)

    outs = []
    for h in range(2):
        sl = slice(h * _HALF, (h + 1) * _HALF)
        ue_t, ie_t = _gather_half(ut_t, it_t, ui[sl], ii[sl])
        outs.append(_mlp_half(ue_t, ie_t, w1u, w1i, b1c, w2t, b2c, w3t, b3c))
    return jnp.concatenate([o.T for o in outs], axis=0)


# speedup vs baseline: 1.0585x; 1.0585x over previous
import functools

import jax
import jax.numpy as jnp
from jax import lax
from jax.experimental import pallas as pl
from jax.experimental.pallas import tpu as pltpu
from jax.experimental.pallas import tpu_sc as plsc

_BATCH = 16384
_D = 32
_H1 = 64
_H2 = 32
_NCLS = 2
_NC = 2
_NS = 16
_NW = _NC * _NS
_BPW = _BATCH // _NW
_G = 8
_NGRP = _BPW // _G
_LANES = 128

_sc_mesh = plsc.VectorSubcoreMesh(core_axis_name="c", subcore_axis_name="s")


@functools.partial(
    pl.kernel,
    mesh=_sc_mesh,
    out_type=(
        jax.ShapeDtypeStruct((_D, _BATCH), jnp.float32),
        jax.ShapeDtypeStruct((_D, _BATCH), jnp.float32),
    ),
    scratch_types=[
        pltpu.VMEM((_BPW + 16,), jnp.int32),
        pltpu.VMEM((2 * _G, _D, _LANES), jnp.float32),
        pltpu.VMEM((_D, _BPW), jnp.float32),
        pltpu.VMEM((_D, _BPW), jnp.float32),
        pltpu.SemaphoreType.DMA,
        pltpu.SemaphoreType.DMA,
    ],
    compiler_params=pltpu.CompilerParams(use_tc_tiling_on_sc=True,
                                         needs_layout_passes=False),
)
def _sc_gather(ut_t, it_t, uidx, iidx, ue_out, ie_out,
               idx_v, blocks, ue_v, ie_v, fsem, ssem):
    wid = lax.axis_index("s") * _NC + lax.axis_index("c")
    base = wid * _BPW
    rows_lo = jnp.arange(16, dtype=jnp.int32)
    rows_hi = rows_lo + 16

    def _fire_one(tab, r, slot):
        start = pl.multiple_of((r >> 7) * _LANES, _LANES)
        pltpu.make_async_copy(
            tab.at[:, pl.ds(start, _LANES)], blocks.at[slot], fsem
        ).start()

    def _extract_one(tab, out_v, r, j, slot):
        pltpu.make_async_copy(
            tab.at[:, pl.ds(0, _LANES)], blocks.at[slot], fsem
        ).wait()
        lvec = jnp.full((16,), r & 127, dtype=jnp.int32)
        jvec = jnp.full((16,), j, dtype=jnp.int32)
        blk = blocks.at[slot]
        v0 = plsc.load_gather(blk, [rows_lo, lvec])
        v1 = plsc.load_gather(blk, [rows_hi, lvec])
        plsc.store_scatter(out_v, [rows_lo, jvec], v0)
        plsc.store_scatter(out_v, [rows_hi, jvec], v1)

    def _run_table(tab, idx_hbm, out_v, out_hbm):
        dst = idx_v.at[pl.ds(0, _BPW)]
        pltpu.make_async_copy(idx_hbm.at[pl.ds(base, _BPW)], dst, ssem).start()
        pltpu.make_async_copy(idx_hbm.at[pl.ds(base, _BPW)], dst, ssem).wait()

        def _fire_group(g, slot0):
            iv = idx_v[pl.ds(g * _G, 16)]
            for kk in range(_G):
                _fire_one(tab, iv[kk], slot0 + kk)

        def _extract_group(g, slot0):
            iv = idx_v[pl.ds(g * _G, 16)]
            for kk in range(_G):
                _extract_one(tab, out_v, iv[kk], g * _G + kk, slot0 + kk)

        _fire_group(0, 0)

        def _body(h, carry):
            g = 2 * h
            _fire_group(g + 1, _G)
            _extract_group(g, 0)

            @pl.when(h < _NGRP // 2 - 1)
            def _():
                _fire_group(g + 2, 0)

            _extract_group(g + 1, _G)
            return carry

        lax.fori_loop(0, _NGRP // 2, _body, 0)
        pltpu.sync_copy(out_v, out_hbm.at[:, pl.ds(base, _BPW)])

    _run_table(ut_t, uidx, ue_v, ue_out)
    _run_table(it_t, iidx, ie_v, ie_out)


_BM = 2048
_GRID = _BATCH // _BM


def _mlp_body(ue, ie, w1u, w1i, b1, w2, b2, w3, b3, out):
    x = jnp.dot(w1u[...], ue[...], preferred_element_type=jnp.float32)
    x = x + jnp.dot(w1i[...], ie[...], preferred_element_type=jnp.float32)
    x = jnp.maximum(x + b1[...], 0.0)
    x = jnp.maximum(jnp.dot(w2[...], x, preferred_element_type=jnp.float32) + b2[...], 0.0)
    x = jnp.maximum(jnp.dot(w3[...], x, preferred_element_type=jnp.float32) + b3[...], 0.0)
    out[...] = x


_mlp = pl.pallas_call(
    _mlp_body,
    grid=(_GRID,),
    in_specs=[
        pl.BlockSpec((_D, _BM), lambda i: (0, i)),
        pl.BlockSpec((_D, _BM), lambda i: (0, i)),
        pl.BlockSpec((_H1, _D), lambda i: (0, 0)),
        pl.BlockSpec((_H1, _D), lambda i: (0, 0)),
        pl.BlockSpec((_H1, 1), lambda i: (0, 0)),
        pl.BlockSpec((_H2, _H1), lambda i: (0, 0)),
        pl.BlockSpec((_H2, 1), lambda i: (0, 0)),
        pl.BlockSpec((_NCLS, _H2), lambda i: (0, 0)),
        pl.BlockSpec((_NCLS, 1), lambda i: (0, 0)),
    ],
    out_specs=pl.BlockSpec((_NCLS, _BM), lambda i: (0, i)),
    out_shape=jax.ShapeDtypeStruct((_NCLS, _BATCH), jnp.float32),
)


def kernel(user_input, item_input, user_table, item_table, W1, b1, W2, b2, W3, b3):
    ue_t, ie_t = _sc_gather(user_table.T, item_table.T,
                            user_input.astype(jnp.int32),
                            item_input.astype(jnp.int32))
    w1t = W1.T
    out_t = _mlp(ue_t, ie_t, w1t[:, :_D], w1t[:, _D:], b1.reshape(_H1, 1),
                 W2.T, b2.reshape(_H2, 1), W3.T, b3.reshape(_NCLS, 1))
    return out_t.T


# speedup vs baseline: 1.0715x; 1.0122x over previous
import functools

import jax
import jax.numpy as jnp
from jax import lax
from jax.experimental import pallas as pl
from jax.experimental.pallas import tpu as pltpu
from jax.experimental.pallas import tpu_sc as plsc

_BATCH = 16384
_D = 32
_H1 = 64
_H2 = 32
_NCLS = 2
_NC = 2
_NS = 16
_NW = _NC * _NS
_BPW = _BATCH // _NW
_G = 8
_NGRP = _BPW // _G
_LANES = 128

_sc_mesh = plsc.VectorSubcoreMesh(core_axis_name="c", subcore_axis_name="s")


@functools.partial(
    pl.kernel,
    mesh=_sc_mesh,
    out_type=(
        jax.ShapeDtypeStruct((_D, _BATCH), jnp.float32),
        jax.ShapeDtypeStruct((_D, _BATCH), jnp.float32),
    ),
    scratch_types=[
        pltpu.VMEM((_BPW + 16,), jnp.int32),
        pltpu.VMEM((2 * _G, _D, _LANES), jnp.float32),
        pltpu.VMEM((_D, _BPW), jnp.float32),
        pltpu.VMEM((_D, _BPW), jnp.float32),
        pltpu.SemaphoreType.DMA,
        pltpu.SemaphoreType.DMA,
    ],
    compiler_params=pltpu.CompilerParams(use_tc_tiling_on_sc=True,
                                         needs_layout_passes=False),
)
def _sc_gather(ut_t, it_t, uidx, iidx, ue_out, ie_out,
               idx_v, blocks, ue_v, ie_v, fsem, ssem):
    wid = lax.axis_index("s") * _NC + lax.axis_index("c")
    base = wid * _BPW
    rows_lo = jnp.arange(16, dtype=jnp.int32)
    rows_hi = rows_lo + 16

    def _fire_one(tab, r, slot):
        start = pl.multiple_of((r >> 7) * _LANES, _LANES)
        pltpu.make_async_copy(
            tab.at[:, pl.ds(start, _LANES)], blocks.at[slot], fsem
        ).start()

    def _extract_one(tab, out_v, r, j, slot):
        pltpu.make_async_copy(
            tab.at[:, pl.ds(0, _LANES)], blocks.at[slot], fsem
        ).wait()
        lvec = jnp.full((16,), r & 127, dtype=jnp.int32)
        jvec = jnp.full((16,), j, dtype=jnp.int32)
        blk = blocks.at[slot]
        v0 = plsc.load_gather(blk, [rows_lo, lvec])
        v1 = plsc.load_gather(blk, [rows_hi, lvec])
        plsc.store_scatter(out_v, [rows_lo, jvec], v0)
        plsc.store_scatter(out_v, [rows_hi, jvec], v1)

    def _run_table(tab, idx_hbm, out_v, out_hbm):
        dst = idx_v.at[pl.ds(0, _BPW)]
        pltpu.make_async_copy(idx_hbm.at[pl.ds(base, _BPW)], dst, ssem).start()
        pltpu.make_async_copy(idx_hbm.at[pl.ds(base, _BPW)], dst, ssem).wait()

        def _fire_group(g, slot0):
            iv = idx_v[pl.ds(g * _G, 16)]
            for kk in range(_G):
                _fire_one(tab, iv[kk], slot0 + kk)

        def _extract_group(g, slot0):
            iv = idx_v[pl.ds(g * _G, 16)]
            for kk in range(_G):
                _extract_one(tab, out_v, iv[kk], g * _G + kk, slot0 + kk)

        _fire_group(0, 0)

        def _body(h, carry):
            g = 2 * h
            _fire_group(g + 1, _G)
            _extract_group(g, 0)

            @pl.when(h < _NGRP // 2 - 1)
            def _():
                _fire_group(g + 2, 0)

            _extract_group(g + 1, _G)
            return carry

        lax.fori_loop(0, _NGRP // 2, _body, 0)
        pltpu.sync_copy(out_v, out_hbm.at[:, pl.ds(base, _BPW)])

    _run_table(ut_t, uidx, ue_v, ue_out)
    _run_table(it_t, iidx, ie_v, ie_out)


_BM = 4096
_GRID = _BATCH // _BM


def _mlp_body(ue, ie, w1u, w1i, b1, w2, b2, w3, b3, out):
    x = jnp.dot(w1u[...], ue[...], preferred_element_type=jnp.float32)
    x = x + jnp.dot(w1i[...], ie[...], preferred_element_type=jnp.float32)
    x = jnp.maximum(x + b1[...], 0.0)
    x = jnp.maximum(jnp.dot(w2[...], x, preferred_element_type=jnp.float32) + b2[...], 0.0)
    x = jnp.maximum(jnp.dot(w3[...], x, preferred_element_type=jnp.float32) + b3[...], 0.0)
    out[...] = x


_mlp = pl.pallas_call(
    _mlp_body,
    grid=(_GRID,),
    in_specs=[
        pl.BlockSpec((_D, _BM), lambda i: (0, i)),
        pl.BlockSpec((_D, _BM), lambda i: (0, i)),
        pl.BlockSpec((_H1, _D), lambda i: (0, 0)),
        pl.BlockSpec((_H1, _D), lambda i: (0, 0)),
        pl.BlockSpec((_H1, 1), lambda i: (0, 0)),
        pl.BlockSpec((_H2, _H1), lambda i: (0, 0)),
        pl.BlockSpec((_H2, 1), lambda i: (0, 0)),
        pl.BlockSpec((_NCLS, _H2), lambda i: (0, 0)),
        pl.BlockSpec((_NCLS, 1), lambda i: (0, 0)),
    ],
    out_specs=pl.BlockSpec((_NCLS, _BM), lambda i: (0, i)),
    out_shape=jax.ShapeDtypeStruct((_NCLS, _BATCH), jnp.float32),
)


def kernel(user_input, item_input, user_table, item_table, W1, b1, W2, b2, W3, b3):
    ue_t, ie_t = _sc_gather(user_table.T, item_table.T,
                            user_input.astype(jnp.int32),
                            item_input.astype(jnp.int32))
    w1t = W1.T
    out_t = _mlp(ue_t, ie_t, w1t[:, :_D], w1t[:, _D:], b1.reshape(_H1, 1),
                 W2.T, b2.reshape(_H2, 1), W3.T, b3.reshape(_NCLS, 1))
    return out_t.T


# speedup vs baseline: 1.0779x; 1.0060x over previous
import functools

import jax
import jax.numpy as jnp
from jax import lax
from jax.experimental import pallas as pl
from jax.experimental.pallas import tpu as pltpu
from jax.experimental.pallas import tpu_sc as plsc

_BATCH = 16384
_D = 32
_H1 = 64
_H2 = 32
_NCLS = 2
_NC = 2
_NS = 16
_NW = _NC * _NS
_BPW = _BATCH // _NW
_G = 8
_NGRP = _BPW // _G
_LANES = 128

_sc_mesh = plsc.VectorSubcoreMesh(core_axis_name="c", subcore_axis_name="s")


@functools.partial(
    pl.kernel,
    mesh=_sc_mesh,
    out_type=(
        jax.ShapeDtypeStruct((_D, _BATCH), jnp.float32),
        jax.ShapeDtypeStruct((_D, _BATCH), jnp.float32),
    ),
    scratch_types=[
        pltpu.VMEM((_BPW + 16,), jnp.int32),
        pltpu.VMEM((2 * _G, _D, _LANES), jnp.float32),
        pltpu.VMEM((_D, _BPW), jnp.float32),
        pltpu.VMEM((_D, _BPW), jnp.float32),
        pltpu.SemaphoreType.DMA,
        pltpu.SemaphoreType.DMA,
    ],
    compiler_params=pltpu.CompilerParams(use_tc_tiling_on_sc=True,
                                         needs_layout_passes=False),
)
def _sc_gather(ut_t, it_t, uidx, iidx, ue_out, ie_out,
               idx_v, blocks, ue_v, ie_v, fsem, ssem):
    wid = lax.axis_index("s") * _NC + lax.axis_index("c")
    base = wid * _BPW
    rows_lo = jnp.arange(16, dtype=jnp.int32)
    rows_hi = rows_lo + 16

    def _fire_one(tab, r, slot):
        start = pl.multiple_of((r >> 7) * _LANES, _LANES)
        pltpu.make_async_copy(
            tab.at[:, pl.ds(start, _LANES)], blocks.at[slot], fsem
        ).start()

    def _extract_one(tab, out_v, r, j, slot):
        pltpu.make_async_copy(
            tab.at[:, pl.ds(0, _LANES)], blocks.at[slot], fsem
        ).wait()
        lvec = jnp.full((16,), r & 127, dtype=jnp.int32)
        jvec = jnp.full((16,), j, dtype=jnp.int32)
        blk = blocks.at[slot]
        v0 = plsc.load_gather(blk, [rows_lo, lvec])
        v1 = plsc.load_gather(blk, [rows_hi, lvec])
        plsc.store_scatter(out_v, [rows_lo, jvec], v0)
        plsc.store_scatter(out_v, [rows_hi, jvec], v1)

    def _run_table(tab, idx_hbm, out_v, out_hbm):
        dst = idx_v.at[pl.ds(0, _BPW)]
        pltpu.make_async_copy(idx_hbm.at[pl.ds(base, _BPW)], dst, ssem).start()
        pltpu.make_async_copy(idx_hbm.at[pl.ds(base, _BPW)], dst, ssem).wait()

        def _fire_group(g, slot0):
            iv = idx_v[pl.ds(g * _G, 16)]
            for kk in range(_G):
                _fire_one(tab, iv[kk], slot0 + kk)

        def _extract_group(g, slot0):
            iv = idx_v[pl.ds(g * _G, 16)]
            for kk in range(_G):
                _extract_one(tab, out_v, iv[kk], g * _G + kk, slot0 + kk)

        _fire_group(0, 0)

        def _body(h, carry):
            g = 2 * h
            _fire_group(g + 1, _G)
            _extract_group(g, 0)

            @pl.when(h < _NGRP // 2 - 1)
            def _():
                _fire_group(g + 2, 0)

            _extract_group(g + 1, _G)
            return carry

        lax.fori_loop(0, _NGRP // 2, _body, 0)
        pltpu.sync_copy(out_v, out_hbm.at[:, pl.ds(base, _BPW)])

    _run_table(ut_t, uidx, ue_v, ue_out)
    _run_table(it_t, iidx, ie_v, ie_out)


_BM = 8192
_GRID = _BATCH // _BM


def _mlp_body(ue, ie, w1u, w1i, b1, w2, b2, w3, b3, out):
    x = jnp.dot(w1u[...], ue[...], preferred_element_type=jnp.float32)
    x = x + jnp.dot(w1i[...], ie[...], preferred_element_type=jnp.float32)
    x = jnp.maximum(x + b1[...], 0.0)
    x = jnp.maximum(jnp.dot(w2[...], x, preferred_element_type=jnp.float32) + b2[...], 0.0)
    x = jnp.maximum(jnp.dot(w3[...], x, preferred_element_type=jnp.float32) + b3[...], 0.0)
    out[...] = x


_mlp = pl.pallas_call(
    _mlp_body,
    grid=(_GRID,),
    in_specs=[
        pl.BlockSpec((_D, _BM), lambda i: (0, i)),
        pl.BlockSpec((_D, _BM), lambda i: (0, i)),
        pl.BlockSpec((_H1, _D), lambda i: (0, 0)),
        pl.BlockSpec((_H1, _D), lambda i: (0, 0)),
        pl.BlockSpec((_H1, 1), lambda i: (0, 0)),
        pl.BlockSpec((_H2, _H1), lambda i: (0, 0)),
        pl.BlockSpec((_H2, 1), lambda i: (0, 0)),
        pl.BlockSpec((_NCLS, _H2), lambda i: (0, 0)),
        pl.BlockSpec((_NCLS, 1), lambda i: (0, 0)),
    ],
    out_specs=pl.BlockSpec((_NCLS, _BM), lambda i: (0, i)),
    out_shape=jax.ShapeDtypeStruct((_NCLS, _BATCH), jnp.float32),
)


def kernel(user_input, item_input, user_table, item_table, W1, b1, W2, b2, W3, b3):
    ue_t, ie_t = _sc_gather(user_table.T, item_table.T,
                            user_input.astype(jnp.int32),
                            item_input.astype(jnp.int32))
    w1t = W1.T
    out_t = _mlp(ue_t, ie_t, w1t[:, :_D], w1t[:, _D:], b1.reshape(_H1, 1),
                 W2.T, b2.reshape(_H2, 1), W3.T, b3.reshape(_NCLS, 1))
    return out_t.T


# speedup vs baseline: 1.0793x; 1.0013x over previous
import functools

import jax
import jax.numpy as jnp
from jax import lax
from jax.experimental import pallas as pl
from jax.experimental.pallas import tpu as pltpu
from jax.experimental.pallas import tpu_sc as plsc

_BATCH = 16384
_D = 32
_H1 = 64
_H2 = 32
_NCLS = 2
_NC = 2
_NS = 16
_NW = _NC * _NS
_BPW = _BATCH // _NW
_G = 8
_NGRP = _BPW // _G
_LANES = 128

_sc_mesh = plsc.VectorSubcoreMesh(core_axis_name="c", subcore_axis_name="s")


@functools.partial(
    pl.kernel,
    mesh=_sc_mesh,
    out_type=(
        jax.ShapeDtypeStruct((_D, _BATCH), jnp.float32),
        jax.ShapeDtypeStruct((_D, _BATCH), jnp.float32),
    ),
    scratch_types=[
        pltpu.VMEM((_BPW + 16,), jnp.int32),
        pltpu.VMEM((2 * _G, _D, _LANES), jnp.float32),
        pltpu.VMEM((_D, _BPW), jnp.float32),
        pltpu.VMEM((_D, _BPW), jnp.float32),
        pltpu.SemaphoreType.DMA,
        pltpu.SemaphoreType.DMA,
    ],
    compiler_params=pltpu.CompilerParams(use_tc_tiling_on_sc=True,
                                         needs_layout_passes=False),
)
def _sc_gather(ut_t, it_t, uidx, iidx, ue_out, ie_out,
               idx_v, blocks, ue_v, ie_v, fsem, ssem):
    wid = lax.axis_index("s") * _NC + lax.axis_index("c")
    base = wid * _BPW
    rows_lo = jnp.arange(16, dtype=jnp.int32)
    rows_hi = rows_lo + 16

    def _fire_one(tab, r, slot):
        start = pl.multiple_of((r >> 7) * _LANES, _LANES)
        pltpu.make_async_copy(
            tab.at[:, pl.ds(start, _LANES)], blocks.at[slot], fsem
        ).start()

    def _extract_one(tab, out_v, r, j, slot):
        pltpu.make_async_copy(
            tab.at[:, pl.ds(0, _LANES)], blocks.at[slot], fsem
        ).wait()
        lvec = jnp.full((16,), r & 127, dtype=jnp.int32)
        jvec = jnp.full((16,), j, dtype=jnp.int32)
        blk = blocks.at[slot]
        v0 = plsc.load_gather(blk, [rows_lo, lvec])
        v1 = plsc.load_gather(blk, [rows_hi, lvec])
        plsc.store_scatter(out_v, [rows_lo, jvec], v0)
        plsc.store_scatter(out_v, [rows_hi, jvec], v1)

    def _run_table(tab, idx_hbm, out_v, out_hbm):
        dst = idx_v.at[pl.ds(0, _BPW)]
        pltpu.make_async_copy(idx_hbm.at[pl.ds(base, _BPW)], dst, ssem).start()
        pltpu.make_async_copy(idx_hbm.at[pl.ds(base, _BPW)], dst, ssem).wait()

        def _fire_group(g, slot0):
            iv = idx_v[pl.ds(g * _G, 16)]
            for kk in range(_G):
                _fire_one(tab, iv[kk], slot0 + kk)

        def _extract_group(g, slot0):
            iv = idx_v[pl.ds(g * _G, 16)]
            for kk in range(_G):
                _extract_one(tab, out_v, iv[kk], g * _G + kk, slot0 + kk)

        _fire_group(0, 0)

        def _body(h, carry):
            g = 2 * h
            _fire_group(g + 1, _G)
            _extract_group(g, 0)

            @pl.when(h < _NGRP // 2 - 1)
            def _():
                _fire_group(g + 2, 0)

            _extract_group(g + 1, _G)
            return carry

        lax.fori_loop(0, _NGRP // 2, _body, 0)
        pltpu.sync_copy(out_v, out_hbm.at[:, pl.ds(base, _BPW)])

    _run_table(ut_t, uidx, ue_v, ue_out)
    _run_table(it_t, iidx, ie_v, ie_out)


_BM = 16384
_GRID = _BATCH // _BM


def _mlp_body(ue, ie, w1u, w1i, b1, w2, b2, w3, b3, out):
    x = jnp.dot(w1u[...], ue[...], preferred_element_type=jnp.float32)
    x = x + jnp.dot(w1i[...], ie[...], preferred_element_type=jnp.float32)
    x = jnp.maximum(x + b1[...], 0.0)
    x = jnp.maximum(jnp.dot(w2[...], x, preferred_element_type=jnp.float32) + b2[...], 0.0)
    x = jnp.maximum(jnp.dot(w3[...], x, preferred_element_type=jnp.float32) + b3[...], 0.0)
    out[...] = x


_mlp = pl.pallas_call(
    _mlp_body,
    grid=(_GRID,),
    in_specs=[
        pl.BlockSpec((_D, _BM), lambda i: (0, i)),
        pl.BlockSpec((_D, _BM), lambda i: (0, i)),
        pl.BlockSpec((_H1, _D), lambda i: (0, 0)),
        pl.BlockSpec((_H1, _D), lambda i: (0, 0)),
        pl.BlockSpec((_H1, 1), lambda i: (0, 0)),
        pl.BlockSpec((_H2, _H1), lambda i: (0, 0)),
        pl.BlockSpec((_H2, 1), lambda i: (0, 0)),
        pl.BlockSpec((_NCLS, _H2), lambda i: (0, 0)),
        pl.BlockSpec((_NCLS, 1), lambda i: (0, 0)),
    ],
    out_specs=pl.BlockSpec((_NCLS, _BM), lambda i: (0, i)),
    out_shape=jax.ShapeDtypeStruct((_NCLS, _BATCH), jnp.float32),
)


def kernel(user_input, item_input, user_table, item_table, W1, b1, W2, b2, W3, b3):
    ue_t, ie_t = _sc_gather(user_table.T, item_table.T,
                            user_input.astype(jnp.int32),
                            item_input.astype(jnp.int32))
    w1t = W1.T
    out_t = _mlp(ue_t, ie_t, w1t[:, :_D], w1t[:, _D:], b1.reshape(_H1, 1),
                 W2.T, b2.reshape(_H2, 1), W3.T, b3.reshape(_NCLS, 1))
    return out_t.T
